# Initial kernel scaffold; baseline (speedup 1.0000x reference)
import jax
import jax.numpy as jnp
from jax.experimental import pallas as pl


def kernel(x, expert_weights, expert_indices, w1, w2):
    raise NotImplementedError("write your pallas kernel here")


# speedup vs baseline: 4.7962x; 4.7962x over previous
import functools

import jax
import jax.numpy as jnp
from jax import lax
from jax.experimental import pallas as pl
from jax.experimental.pallas import tpu as pltpu

SEQ = 2048
D_MODEL = 768
D_FF = 1024
NUM_EXPERTS = 64
BT = 128


def _mlp_kernel(offs_ref, cnts_ref, xs_ref, ws_ref, w1_ref, w2_ref, ys_ref):
    e = pl.program_id(0)
    start = offs_ref[e]
    cnt = cnts_ref[e]
    t0 = start // BT
    t1 = lax.div(start + cnt + BT - 1, BT)
    w1e = w1_ref[0]
    w2e = w2_ref[0]

    def body(j, _):
        s = (t0 + j) * BT
        xb = xs_ref[pl.ds(s, BT), :]
        h = jnp.dot(xb, w1e, preferred_element_type=jnp.float32)
        h = h * jax.nn.sigmoid(h)
        yb = jnp.dot(h, w2e, preferred_element_type=jnp.float32)
        yb = yb * ws_ref[pl.ds(s, BT), :]
        rid = s + lax.broadcasted_iota(jnp.int32, (BT, 1), 0)
        mask = (rid >= start) & (rid < start + cnt)
        ys_ref[pl.ds(s, BT), :] = jnp.where(mask, yb, ys_ref[pl.ds(s, BT), :])
        return 0

    lax.fori_loop(0, t1 - t0, body, 0)


def _grouped_mlp(xs, ws, offsets, counts, w1, w2):
    grid_spec = pltpu.PrefetchScalarGridSpec(
        num_scalar_prefetch=2,
        grid=(NUM_EXPERTS,),
        in_specs=[
            pl.BlockSpec((SEQ, D_MODEL), lambda e, o, c: (0, 0)),
            pl.BlockSpec((SEQ, 1), lambda e, o, c: (0, 0)),
            pl.BlockSpec((1, D_MODEL, D_FF), lambda e, o, c: (e, 0, 0)),
            pl.BlockSpec((1, D_FF, D_MODEL), lambda e, o, c: (e, 0, 0)),
        ],
        out_specs=pl.BlockSpec((SEQ, D_MODEL), lambda e, o, c: (0, 0)),
    )
    return pl.pallas_call(
        _mlp_kernel,
        grid_spec=grid_spec,
        out_shape=jax.ShapeDtypeStruct((SEQ, D_MODEL), jnp.float32),
        compiler_params=pltpu.CompilerParams(
            dimension_semantics=("arbitrary",),
        ),
    )(offsets, counts, xs, ws, w1, w2)


def kernel(x, expert_weights, expert_indices, w1, w2):
    flat_x = x.reshape(SEQ, D_MODEL)
    idx = expert_indices.reshape(SEQ).astype(jnp.int32)
    wflat = expert_weights.reshape(SEQ)

    counts = jnp.bincount(idx, length=NUM_EXPERTS).astype(jnp.int32)
    offsets = (jnp.cumsum(counts) - counts).astype(jnp.int32)
    order = jnp.argsort(idx)
    xs = flat_x[order]
    ws = wflat[order][:, None]

    ys = _grouped_mlp(xs, ws, offsets, counts, w1, w2)

    out_flat = jnp.zeros_like(flat_x).at[order].set(ys)
    return out_flat.reshape(x.shape), counts
